# Initial kernel scaffold; baseline (speedup 1.0000x reference)
import jax
import jax.numpy as jnp
from jax.experimental import pallas as pl


def kernel(points, W1, b1, W2, b2, Wa1, aa1, Wa2, aa2, Wd, bd, Wu1, bu1, Wu2, bu2, Wl1, bl1, gamma, beta, Wl2, bl2):
    raise NotImplementedError("write your pallas kernel here")


# speedup vs baseline: 4.1894x; 4.1894x over previous
import functools

import jax
import jax.numpy as jnp
from jax import lax
from jax.experimental import pallas as pl
from jax.experimental.pallas import tpu as pltpu
from jax.experimental.pallas import tpu_sc as plsc

K = 16
_NW = 32


def _knn_pallas(q, pT, block):
    nq = q.shape[0]
    n = pT.shape[1]

    def kern(q_ref, pT_ref, o_ref):
        qb = q_ref[...]
        pt = pT_ref[...]
        r2 = jnp.sum(pt * pt, axis=0, keepdims=True)
        qq = jnp.sum(qb * qb, axis=1, keepdims=True)
        d = qq - 2.0 * jnp.dot(qb, pt, preferred_element_type=jnp.float32) + r2
        cols = lax.broadcasted_iota(jnp.int32, (block, n), 1)
        outs = []
        for _ in range(K):
            m = jnp.min(d, axis=1, keepdims=True)
            j = jnp.min(jnp.where(d == m, cols, jnp.int32(1 << 30)), axis=1)
            outs.append(j)
            d = jnp.where(cols == j[:, None], jnp.float32(jnp.inf), d)
        o_ref[...] = jnp.stack(outs, axis=1)

    return pl.pallas_call(
        kern,
        grid=(nq // block,),
        in_specs=[
            pl.BlockSpec((block, q.shape[1]), lambda i: (i, 0)),
            pl.BlockSpec(pT.shape, lambda i: (0, 0)),
        ],
        out_specs=pl.BlockSpec((block, K), lambda i: (i, 0)),
        out_shape=jax.ShapeDtypeStruct((nq, K), jnp.int32),
    )(q, pT)


def _mm(x, W, b, y=None, Wy=None, block=400):
    R, Din = x.shape
    Dout = W.shape[1]

    def kern(*refs):
        if y is not None:
            x_ref, W_ref, b_ref, y_ref, Wy_ref, o_ref = refs
        else:
            x_ref, W_ref, b_ref, o_ref = refs
        acc = jnp.dot(x_ref[...], W_ref[...], preferred_element_type=jnp.float32)
        if y is not None:
            acc = acc + jnp.dot(y_ref[...], Wy_ref[...],
                                preferred_element_type=jnp.float32)
        o_ref[...] = acc + b_ref[...]

    in_specs = [
        pl.BlockSpec((block, Din), lambda i: (i, 0)),
        pl.BlockSpec((Din, Dout), lambda i: (0, 0)),
        pl.BlockSpec((1, Dout), lambda i: (0, 0)),
    ]
    args = [x, W, b.reshape(1, Dout)]
    if y is not None:
        in_specs += [
            pl.BlockSpec((block, y.shape[1]), lambda i: (i, 0)),
            pl.BlockSpec((y.shape[1], Dout), lambda i: (0, 0)),
        ]
        args += [y, Wy]
    return pl.pallas_call(
        kern,
        grid=(R // block,),
        in_specs=in_specs,
        out_specs=pl.BlockSpec((block, Dout), lambda i: (i, 0)),
        out_shape=jax.ShapeDtypeStruct((R, Dout), jnp.float32),
    )(*args)


def _sc_gather(table, idx):
    N, D = table.shape
    M = idx.shape[0]
    C = 128
    assert M % (C * _NW) == 0 and D % 128 == 0
    b_per_w = M // _NW
    T = b_per_w // C
    mesh = plsc.VectorSubcoreMesh(core_axis_name="c", subcore_axis_name="s")

    @functools.partial(
        pl.kernel,
        mesh=mesh,
        out_type=jax.ShapeDtypeStruct((M, D), jnp.float32),
        scratch_types=[
            pltpu.VMEM((C,), jnp.int32),
            pltpu.VMEM((C, D), jnp.float32),
            pltpu.SemaphoreType.DMA,
        ],
    )
    def k(table_hbm, idx_hbm, out_hbm, idx_v, rows_v, sem):
        wid = lax.axis_index("s") * 2 + lax.axis_index("c")
        base = wid * b_per_w

        def body(t, carry):
            off = base + t * C
            pltpu.sync_copy(idx_hbm.at[pl.ds(off, C)], idx_v)
            pltpu.async_copy(table_hbm.at[idx_v], rows_v, sem).wait()
            pltpu.sync_copy(rows_v, out_hbm.at[pl.ds(off, C)])
            return carry

        lax.fori_loop(0, T, body, 0)

    return k(table, idx)


def _ec_post(U, G, block=400):
    R, D = U.shape

    def kern(U_ref, G_ref, o_ref):
        G3 = G_ref[...].reshape(block, K, D)
        h = jnp.maximum(U_ref[...][:, None, :] + G3, 0.0)
        o_ref[...] = jnp.max(h, axis=1)

    return pl.pallas_call(
        kern,
        grid=(R // block,),
        in_specs=[
            pl.BlockSpec((block, D), lambda i: (i, 0)),
            pl.BlockSpec((block * K, D), lambda i: (i, 0)),
        ],
        out_specs=pl.BlockSpec((block, D), lambda i: (i, 0)),
        out_shape=jax.ShapeDtypeStruct((R, D), jnp.float32),
    )(U, G)


def _gat(h, G, a_first, a_second, block=400):
    R, D = h.shape

    def kern(h_ref, G_ref, af_ref, as_ref, o_ref):
        hb = h_ref[...]
        G3 = G_ref[...].reshape(block, K, D)
        s1 = jnp.sum(hb * af_ref[...], axis=1, keepdims=True)
        s2 = jnp.sum(G3 * as_ref[...][None], axis=2, keepdims=True)
        e = s1[:, None, :] + s2
        e = jnp.where(e >= 0, e, 0.2 * e)
        e = e - jnp.max(e, axis=1, keepdims=True)
        w = jnp.exp(e)
        alpha = w / jnp.sum(w, axis=1, keepdims=True)
        out = jnp.sum(G3 * alpha, axis=1)
        o_ref[...] = jnp.where(out > 0, out, jnp.exp(out) - 1.0)

    return pl.pallas_call(
        kern,
        grid=(R // block,),
        in_specs=[
            pl.BlockSpec((block, D), lambda i: (i, 0)),
            pl.BlockSpec((block * K, D), lambda i: (i, 0)),
            pl.BlockSpec((1, D), lambda i: (0, 0)),
            pl.BlockSpec((1, D), lambda i: (0, 0)),
        ],
        out_specs=pl.BlockSpec((block, D), lambda i: (i, 0)),
        out_shape=jax.ShapeDtypeStruct((R, D), jnp.float32),
    )(h, G, a_first, a_second)


def _final(x5, Wu1, bu1, Wu2, bu2, Wl1, bl1, gamma, beta, Wl2, bl2, nvalid):
    R, D = x5.shape

    def kern(x_ref, Wu1r, bu1r, Wu2r, bu2r, Wl1r, bl1r, gr, br, Wl2r, bl2r,
             o_ref):
        x = x_ref[...]
        rows = lax.broadcasted_iota(jnp.int32, (R, 1), 0)
        xm = jnp.where(rows < nvalid, x, 0.0)
        mu = jnp.sum(xm, axis=0, keepdims=True) / float(nvalid)
        y1 = jnp.maximum(
            jnp.dot(x, Wu1r[...], preferred_element_type=jnp.float32)
            + bu1r[...], 0.0)
        y2 = jnp.maximum(
            jnp.dot(mu, Wu2r[...], preferred_element_type=jnp.float32)
            + bu2r[...], 0.0)
        x6 = y1 + y2
        hh = jnp.dot(x6, Wl1r[...], preferred_element_type=jnp.float32) + bl1r[...]
        hh = jnp.maximum(gr[...] * hh + br[...], 0.0)
        o_ref[...] = (jnp.dot(hh, Wl2r[...], preferred_element_type=jnp.float32)
                      + bl2r[...])

    mats = [x5, Wu1, bu1.reshape(1, -1), Wu2, bu2.reshape(1, -1),
            Wl1, bl1.reshape(1, -1), gamma.reshape(1, -1), beta.reshape(1, -1),
            Wl2, bl2.reshape(1, -1)]
    in_specs = [pl.BlockSpec(m.shape, lambda i, nd=m.ndim: (0,) * nd)
                for m in mats]
    return pl.pallas_call(
        kern,
        grid=(1,),
        in_specs=in_specs,
        out_specs=pl.BlockSpec((R, Wl2.shape[1]), lambda i: (0, 0)),
        out_shape=jax.ShapeDtypeStruct((R, Wl2.shape[1]), jnp.float32),
    )(*mats)


def kernel(points, W1, b1, W2, b2, Wa1, aa1, Wa2, aa2, Wd, bd, Wu1, bu1,
           Wu2, bu2, Wl1, bl1, gamma, beta, Wl2, bl2):
    p = points[0]
    pT = p.T
    z64 = jnp.zeros((64,), jnp.float32)
    z128 = jnp.zeros((128,), jnp.float32)
    z256 = jnp.zeros((256,), jnp.float32)
    z512 = jnp.zeros((512,), jnp.float32)

    idx1 = _knn_pallas(p, pT, block=400)
    i1f = jnp.concatenate(
        [idx1.reshape(-1), jnp.zeros((3840,), jnp.int32)])

    zpad = jnp.zeros((3, 64), jnp.float32)
    W1a = jnp.concatenate([W1[:3], zpad], axis=1)
    W1b = jnp.concatenate([W1[3:], zpad], axis=1)
    b1p = jnp.concatenate([b1, z64])
    U1 = _mm(p, W1a - W1b, b1p)
    V1 = _mm(p, W1b, z128)
    x1 = _ec_post(U1, _sc_gather(V1, i1f))

    zpad2 = jnp.zeros((64, 128), jnp.float32)
    W2a = jnp.concatenate([W2[:64], zpad2], axis=0)
    W2b = jnp.concatenate([W2[64:], zpad2], axis=0)
    U2 = _mm(x1, W2a - W2b, b2)
    V2 = _mm(x1, W2b, z128)
    x2 = _ec_post(U2, _sc_gather(V2, i1f))

    h1 = _mm(x2, Wa1, z256)
    x3 = _gat(h1, _sc_gather(h1, i1f),
              aa1[:256].reshape(1, 256), aa1[256:].reshape(1, 256))

    h2 = _mm(x3, Wa2, z512)
    x4 = _gat(h2, _sc_gather(h2, i1f),
              aa2[:512].reshape(1, 512), aa2[512:].reshape(1, 512))

    pn = p[0::4]
    pn_pad = jnp.concatenate(
        [pn, jnp.broadcast_to(pn[:1], (60, 3))], axis=0)
    idx2 = _knn_pallas(pn_pad, pT, block=320)
    i2f = idx2.reshape(-1)
    Wd_p, Wd_x = Wd[:3], Wd[3:]
    Mtd = _mm(x4, Wd_x, z512, y=p, Wy=Wd_p)
    Qtd = _mm(pn_pad, -Wd_p, bd, block=320)
    x5 = _ec_post(Qtd, _sc_gather(Mtd, i2f), block=320)

    out = _final(x5, Wu1, bu1, Wu2, bu2, Wl1, bl1, gamma, beta, Wl2, bl2,
                 nvalid=2500)
    return out[:2500]
